# Initial kernel scaffold; baseline (speedup 1.0000x reference)
import jax
import jax.numpy as jnp
from jax.experimental import pallas as pl


def kernel(input_hidden_states, output_hidden_states, router_hidden_states, W_router, W_down, W_up):
    raise NotImplementedError("write your pallas kernel here")


# speedup vs baseline: 2.7571x; 2.7571x over previous
import jax
import jax.numpy as jnp
from jax.experimental import pallas as pl
from jax.experimental.pallas import tpu as pltpu

_B = 2
_S = 2048
_H = 2048
_E = 8
_A = 128
_T = _B * _S
_TM = 256


def _moe_tile_kernel(xi_ref, xo_ref, xr_ref, wr_ref, wd_ref, wu_ref,
                     out_ref, logits_ref):
    logits = jax.lax.dot_general(
        xr_ref[...], wr_ref[...],
        dimension_numbers=(((1,), (1,)), ((), ())),
        preferred_element_type=jnp.float32)
    logits_ref[...] = logits

    p = jax.nn.softmax(logits, axis=-1)
    idx = jax.lax.broadcasted_iota(jnp.int32, (_TM, _E), 1)
    p1 = jnp.max(p, axis=-1, keepdims=True)
    i1 = jnp.min(jnp.where(p == p1, idx, _E), axis=-1, keepdims=True)
    sel1 = idx == i1
    pm = jnp.where(sel1, -jnp.inf, p)
    p2 = jnp.max(pm, axis=-1, keepdims=True)
    i2 = jnp.min(jnp.where(pm == p2, idx, _E), axis=-1, keepdims=True)
    sel2 = idx == i2
    denom = p1 + p2
    w = (jnp.where(sel1, p, 0.0) + jnp.where(sel2, p, 0.0)) / denom

    h = jax.lax.dot_general(
        xi_ref[...], wd_ref[...],
        dimension_numbers=(((1,), (1,)), ((), ())),
        preferred_element_type=jnp.float32)
    h = jax.nn.gelu(h)
    h = (h.reshape(_TM, _E, _A) * w[:, :, None]).reshape(_TM, _E * _A)
    h = h.astype(jnp.bfloat16)
    out = jax.lax.dot_general(
        h, wu_ref[...],
        dimension_numbers=(((1,), (0,)), ((), ())),
        preferred_element_type=jnp.float32)
    out_ref[...] = out + xo_ref[...]


def kernel(input_hidden_states, output_hidden_states, router_hidden_states,
           W_router, W_down, W_up):
    orig_shape = output_hidden_states.shape
    xi = input_hidden_states.reshape(_T, _H).astype(jnp.bfloat16)
    xo = output_hidden_states.reshape(_T, _H)
    xr = router_hidden_states.reshape(_T, _H).astype(jnp.bfloat16)
    wd = W_down.reshape(_E * _A, _H).astype(jnp.bfloat16)
    wu = W_up.transpose(0, 2, 1).reshape(_E * _A, _H).astype(jnp.bfloat16)

    grid = (_T // _TM,)
    out, logits = pl.pallas_call(
        _moe_tile_kernel,
        grid=grid,
        in_specs=[
            pl.BlockSpec((_TM, _H), lambda i: (i, 0)),
            pl.BlockSpec((_TM, _H), lambda i: (i, 0)),
            pl.BlockSpec((_TM, _H), lambda i: (i, 0)),
            pl.BlockSpec((_E, _H), lambda i: (0, 0)),
            pl.BlockSpec((_E * _A, _H), lambda i: (0, 0)),
            pl.BlockSpec((_E * _A, _H), lambda i: (0, 0)),
        ],
        out_specs=[
            pl.BlockSpec((_TM, _H), lambda i: (i, 0)),
            pl.BlockSpec((_TM, _E), lambda i: (i, 0)),
        ],
        out_shape=[
            jax.ShapeDtypeStruct((_T, _H), jnp.float32),
            jax.ShapeDtypeStruct((_T, _E), jnp.float32),
        ],
        compiler_params=pltpu.CompilerParams(
            dimension_semantics=("arbitrary",),
        ),
    )(xi, xo, xr, W_router, wd, wu)

    return out.reshape(orig_shape), logits


# speedup vs baseline: 4.0531x; 1.4701x over previous
import jax
import jax.numpy as jnp
from jax.experimental import pallas as pl
from jax.experimental.pallas import tpu as pltpu

_B = 2
_S = 2048
_H = 2048
_E = 8
_A = 128
_T = _B * _S
_TM = 512


def _moe_tile_kernel(xi_ref, xo_ref, xr_ref, wr_ref, wd_ref, wu_ref,
                     out_ref, logits_ref):
    xr = xr_ref[...].astype(jnp.bfloat16)
    logits = jax.lax.dot_general(
        xr, wr_ref[...],
        dimension_numbers=(((1,), (1,)), ((), ())),
        preferred_element_type=jnp.float32)
    logits_ref[...] = logits

    p = jax.nn.softmax(logits, axis=-1)
    idx = jax.lax.broadcasted_iota(jnp.int32, (_TM, _E), 1)
    p1 = jnp.max(p, axis=-1, keepdims=True)
    i1 = jnp.min(jnp.where(p == p1, idx, _E), axis=-1, keepdims=True)
    sel1 = idx == i1
    pm = jnp.where(sel1, -jnp.inf, p)
    p2 = jnp.max(pm, axis=-1, keepdims=True)
    i2 = jnp.min(jnp.where(pm == p2, idx, _E), axis=-1, keepdims=True)
    sel2 = idx == i2
    denom = p1 + p2
    w = (jnp.where(sel1, p, 0.0) + jnp.where(sel2, p, 0.0)) / denom

    lane = jax.lax.broadcasted_iota(jnp.int32, (_E, _E * _A), 1)
    row = jax.lax.broadcasted_iota(jnp.int32, (_E, _E * _A), 0)
    blockmat = (lane // _A == row).astype(jnp.bfloat16)
    scale = jax.lax.dot_general(
        w.astype(jnp.bfloat16), blockmat,
        dimension_numbers=(((1,), (0,)), ((), ())),
        preferred_element_type=jnp.float32)

    h = jax.lax.dot_general(
        xi_ref[...].astype(jnp.bfloat16), wd_ref[...],
        dimension_numbers=(((1,), (1,)), ((), ())),
        preferred_element_type=jnp.float32)
    h = jax.nn.gelu(h)
    h = (h * scale).astype(jnp.bfloat16)
    out = jax.lax.dot_general(
        h, wu_ref[...],
        dimension_numbers=(((1,), (0,)), ((), ())),
        preferred_element_type=jnp.float32)
    out_ref[...] = out + xo_ref[...]


def kernel(input_hidden_states, output_hidden_states, router_hidden_states,
           W_router, W_down, W_up):
    orig_shape = output_hidden_states.shape
    xi = input_hidden_states.reshape(_T, _H)
    xo = output_hidden_states.reshape(_T, _H)
    xr = router_hidden_states.reshape(_T, _H)
    wd = W_down.reshape(_E * _A, _H).astype(jnp.bfloat16)
    wu = W_up.transpose(0, 2, 1).reshape(_E * _A, _H).astype(jnp.bfloat16)

    grid = (_T // _TM,)
    out, logits = pl.pallas_call(
        _moe_tile_kernel,
        grid=grid,
        in_specs=[
            pl.BlockSpec((_TM, _H), lambda i: (i, 0)),
            pl.BlockSpec((_TM, _H), lambda i: (i, 0)),
            pl.BlockSpec((_TM, _H), lambda i: (i, 0)),
            pl.BlockSpec((_E, _H), lambda i: (0, 0)),
            pl.BlockSpec((_E * _A, _H), lambda i: (0, 0)),
            pl.BlockSpec((_E * _A, _H), lambda i: (0, 0)),
        ],
        out_specs=[
            pl.BlockSpec((_TM, _H), lambda i: (i, 0)),
            pl.BlockSpec((_TM, _E), lambda i: (i, 0)),
        ],
        out_shape=[
            jax.ShapeDtypeStruct((_T, _H), jnp.float32),
            jax.ShapeDtypeStruct((_T, _E), jnp.float32),
        ],
        compiler_params=pltpu.CompilerParams(
            dimension_semantics=("arbitrary",),
        ),
    )(xi, xo, xr, W_router, wd, wu)

    return out.reshape(orig_shape), logits
